# baseline (speedup 1.0000x reference)
import functools

import jax
import jax.numpy as jnp
from jax import lax
from jax.experimental import pallas as pl
from jax.experimental.pallas import tpu as pltpu
from jax.experimental.pallas import tpu_sc as plsc

B, L, V, E, H, OUT = 64, 200, 100000, 128, 512, 2
G4 = 4 * H
TS = 10
NB = L // TS


def _sc_gather(table, idx):
    n = idx.shape[0]
    d = table.shape[1]
    info = plsc.get_sparse_core_info()
    nw = info.num_cores * info.num_subcores
    n_per_w = n // nw

    mesh = plsc.VectorSubcoreMesh(core_axis_name="c", subcore_axis_name="s")

    @functools.partial(
        pl.kernel,
        mesh=mesh,
        out_type=jax.ShapeDtypeStruct((n, d), jnp.float32),
        scratch_types=[
            pltpu.VMEM((n_per_w,), jnp.int32),
            pltpu.VMEM((n_per_w, d), jnp.float32),
            pltpu.SemaphoreType.DMA,
        ],
    )
    def gath(table_hbm, idx_hbm, out_hbm, idx_v, rows_v, sem):
        wid = lax.axis_index("s") * info.num_cores + lax.axis_index("c")
        base = wid * n_per_w
        pltpu.sync_copy(idx_hbm.at[pl.ds(base, n_per_w)], idx_v)
        pltpu.async_copy(table_hbm.at[idx_v], rows_v, sem).wait()
        pltpu.sync_copy(rows_v, out_hbm.at[pl.ds(base, n_per_w)])

    return gath(table, idx)


def _lstm_step(g_ref, wh_ref, h_s, c_s, o_ref):
    hb = h_s[...].astype(jnp.bfloat16)

    def gate(k):
        return g_ref[:, k * H:(k + 1) * H].astype(jnp.float32) + (
            lax.dot_general(hb, wh_ref[k * H:(k + 1) * H, :],
                            (((1,), (1,)), ((), ())),
                            preferred_element_type=jnp.float32))

    i = jax.nn.sigmoid(gate(0))
    f = jax.nn.sigmoid(gate(1))
    gg = jnp.tanh(gate(2))
    o = jax.nn.sigmoid(gate(3))
    c = f * c_s[...] + i * gg
    h = o * jnp.tanh(c)
    c_s[...] = c
    h_s[...] = h
    o_ref[...] = h.astype(o_ref.dtype)


def _rec0_body(xf_ref, xr_ref, wi_f_ref, wi_r_ref, whf_ref, whr_ref,
               bf_ref, br_ref, of_ref, or_ref, gf, gr, hf, cf, hr, cr):
    t = pl.program_id(0)

    @pl.when(t == 0)
    def _():
        hf[...] = jnp.zeros_like(hf)
        cf[...] = jnp.zeros_like(cf)
        hr[...] = jnp.zeros_like(hr)
        cr[...] = jnp.zeros_like(cr)

    gf[...] = (jnp.dot(xf_ref[...].reshape(TS * B, E), wi_f_ref[...],
                       preferred_element_type=jnp.float32)
               + bf_ref[...]).astype(gf.dtype)
    gr[...] = (jnp.dot(xr_ref[...].reshape(TS * B, E), wi_r_ref[...],
                       preferred_element_type=jnp.float32)
               + br_ref[...]).astype(gr.dtype)

    for j in range(TS):
        _lstm_step(gf.at[pl.ds(j * B, B)], whf_ref, hf, cf, of_ref.at[j])
        jr = TS - 1 - j
        _lstm_step(gr.at[pl.ds(jr * B, B)], whr_ref, hr, cr,
                   or_ref.at[jr])


def _rec0(x, wi_f, wi_r, whh_f, whh_r, bias_f, bias_r):
    return pl.pallas_call(
        _rec0_body,
        grid=(NB,),
        in_specs=[
            pl.BlockSpec((TS, B, E), lambda t: (t, 0, 0)),
            pl.BlockSpec((TS, B, E), lambda t: (NB - 1 - t, 0, 0)),
            pl.BlockSpec((E, G4), lambda t: (0, 0)),
            pl.BlockSpec((E, G4), lambda t: (0, 0)),
            pl.BlockSpec((G4, H), lambda t: (0, 0)),
            pl.BlockSpec((G4, H), lambda t: (0, 0)),
            pl.BlockSpec((1, G4), lambda t: (0, 0)),
            pl.BlockSpec((1, G4), lambda t: (0, 0)),
        ],
        out_specs=[
            pl.BlockSpec((TS, B, H), lambda t: (t, 0, 0)),
            pl.BlockSpec((TS, B, H), lambda t: (NB - 1 - t, 0, 0)),
        ],
        out_shape=[
            jax.ShapeDtypeStruct((L, B, H), jnp.bfloat16),
            jax.ShapeDtypeStruct((L, B, H), jnp.bfloat16),
        ],
        scratch_shapes=[
            pltpu.VMEM((TS * B, G4), jnp.bfloat16),
            pltpu.VMEM((TS * B, G4), jnp.bfloat16),
            pltpu.VMEM((B, H), jnp.float32),
            pltpu.VMEM((B, H), jnp.float32),
            pltpu.VMEM((B, H), jnp.float32),
            pltpu.VMEM((B, H), jnp.float32),
        ],
        compiler_params=pltpu.CompilerParams(
            dimension_semantics=("arbitrary",),
        ),
    )(x, x, wi_f, wi_r, whh_f, whh_r, bias_f, bias_r)


def _g1_block(a_ref, b_ref, wa_ref, wb_ref, bias_ref):
    return (
        jnp.dot(a_ref[...].reshape(TS * B, H), wa_ref[...],
                preferred_element_type=jnp.float32)
        + jnp.dot(b_ref[...].reshape(TS * B, H), wb_ref[...],
                  preferred_element_type=jnp.float32)
        + bias_ref[...]).astype(jnp.bfloat16)


def _rec1_body(af_n_ref, bf_n_ref, ar_n_ref, br_n_ref,
               af0_ref, bf0_ref, ar0_ref, br0_ref,
               wia_f_ref, wib_f_ref, wia_r_ref, wib_r_ref,
               whf_ref, whr_ref, bf_ref, br_ref,
               of_ref, or_ref, gf, gr, hf, cf, hr, cr):
    t = pl.program_id(0)

    @pl.when(t == 0)
    def _():
        hf[...] = jnp.zeros_like(hf)
        cf[...] = jnp.zeros_like(cf)
        hr[...] = jnp.zeros_like(hr)
        cr[...] = jnp.zeros_like(cr)
        gf[0] = _g1_block(af0_ref, bf0_ref, wia_f_ref, wib_f_ref, bf_ref)
        gr[0] = _g1_block(ar0_ref, br0_ref, wia_r_ref, wib_r_ref, br_ref)

    @pl.when(t < NB - 1)
    def _():
        nxt = (t + 1) % 2
        gf[nxt] = _g1_block(af_n_ref, bf_n_ref, wia_f_ref, wib_f_ref,
                            bf_ref)
        gr[nxt] = _g1_block(ar_n_ref, br_n_ref, wia_r_ref, wib_r_ref,
                            br_ref)

    cur = t % 2
    for j in range(TS):
        _lstm_step(gf.at[cur, pl.ds(j * B, B)], whf_ref, hf, cf,
                   of_ref.at[j])
        jr = TS - 1 - j
        _lstm_step(gr.at[cur, pl.ds(jr * B, B)], whr_ref, hr, cr,
                   or_ref.at[jr])


def _rec1(h0f, h0r, wia_f, wib_f, wia_r, wib_r, whh_f, whh_r,
          bias_f, bias_r):
    def nxt(t):
        return jnp.minimum(t + 1, NB - 1)

    blk_n = pl.BlockSpec((TS, B, H), lambda t: (nxt(t), 0, 0))
    blk_rev_n = pl.BlockSpec((TS, B, H), lambda t: (NB - 1 - nxt(t), 0, 0))
    blk0 = pl.BlockSpec((TS, B, H), lambda t: (0, 0, 0))
    blk_rev0 = pl.BlockSpec((TS, B, H), lambda t: (NB - 1, 0, 0))
    wspec = pl.BlockSpec((H, G4), lambda t: (0, 0))
    return pl.pallas_call(
        _rec1_body,
        grid=(NB,),
        in_specs=[
            blk_n, blk_n, blk_rev_n, blk_rev_n,
            blk0, blk0, blk_rev0, blk_rev0,
            wspec, wspec, wspec, wspec,
            pl.BlockSpec((G4, H), lambda t: (0, 0)),
            pl.BlockSpec((G4, H), lambda t: (0, 0)),
            pl.BlockSpec((1, G4), lambda t: (0, 0)),
            pl.BlockSpec((1, G4), lambda t: (0, 0)),
        ],
        out_specs=[
            pl.BlockSpec((TS, B, H), lambda t: (t, 0, 0)),
            pl.BlockSpec((TS, B, H), lambda t: (NB - 1 - t, 0, 0)),
        ],
        out_shape=[
            jax.ShapeDtypeStruct((L, B, H), jnp.bfloat16),
            jax.ShapeDtypeStruct((L, B, H), jnp.bfloat16),
        ],
        scratch_shapes=[
            pltpu.VMEM((2, TS * B, G4), jnp.bfloat16),
            pltpu.VMEM((2, TS * B, G4), jnp.bfloat16),
            pltpu.VMEM((B, H), jnp.float32),
            pltpu.VMEM((B, H), jnp.float32),
            pltpu.VMEM((B, H), jnp.float32),
            pltpu.VMEM((B, H), jnp.float32),
        ],
        compiler_params=pltpu.CompilerParams(
            dimension_semantics=("arbitrary",),
        ),
    )(h0f, h0r, h0f, h0r, h0f, h0r, h0f, h0r,
      wia_f, wib_f, wia_r, wib_r, whh_f, whh_r, bias_f, bias_r)


def _attn_body(xf_ref, xr_ref, wa_ref, ba_ref, wf_ref, bf_ref,
               out_ref, aw_ref):
    bb = xf_ref.shape[1]
    xf = xf_ref[...].astype(jnp.float32)
    xr = xr_ref[...].astype(jnp.float32)
    wa = wa_ref[...]
    lg = (
        jnp.dot(xf.reshape(L * bb, H), wa[:, :H].T,
                preferred_element_type=jnp.float32)
        + jnp.dot(xr.reshape(L * bb, H), wa[:, H:].T,
                  preferred_element_type=jnp.float32)
    ).reshape(L, bb) + ba_ref[0, 0]
    m = jnp.max(lg, axis=0, keepdims=True)
    e = jnp.exp(lg - m)
    w = e / jnp.sum(e, axis=0, keepdims=True)
    aw_ref[...] = w.T
    ctx_f = jnp.sum(w[:, :, None] * xf, axis=0)
    ctx_r = jnp.sum(w[:, :, None] * xr, axis=0)
    wf = wf_ref[...]
    out_ref[...] = (
        jnp.dot(ctx_f, wf[:, :H].T, preferred_element_type=jnp.float32)
        + jnp.dot(ctx_r, wf[:, H:].T, preferred_element_type=jnp.float32)
        + bf_ref[...]
    )


def _attn(h_f, h_r, wa, ba, wf, bf, bb=16):
    return pl.pallas_call(
        _attn_body,
        grid=(B // bb,),
        in_specs=[
            pl.BlockSpec((L, bb, H), lambda b: (0, b, 0)),
            pl.BlockSpec((L, bb, H), lambda b: (0, b, 0)),
            pl.BlockSpec((1, 2 * H), lambda b: (0, 0)),
            pl.BlockSpec((1, 1), lambda b: (0, 0)),
            pl.BlockSpec((OUT, 2 * H), lambda b: (0, 0)),
            pl.BlockSpec((1, OUT), lambda b: (0, 0)),
        ],
        out_specs=[
            pl.BlockSpec((bb, OUT), lambda b: (b, 0)),
            pl.BlockSpec((bb, L), lambda b: (b, 0)),
        ],
        out_shape=[
            jax.ShapeDtypeStruct((B, OUT), jnp.float32),
            jax.ShapeDtypeStruct((B, L), jnp.float32),
        ],
        compiler_params=pltpu.CompilerParams(
            dimension_semantics=("parallel",),
        ),
    )(h_f, h_r, wa, ba.reshape(1, 1), wf, bf.reshape(1, OUT))


def kernel(text, wih_0f, whh_0f, bih_0f, bhh_0f, wih_0r, whh_0r, bih_0r,
           bhh_0r, wih_1f, whh_1f, bih_1f, bhh_1f, wih_1r, whh_1r, bih_1r,
           bhh_1r, emb, wa, ba, wf, bf):
    bf16 = jnp.bfloat16
    idx = text.T.reshape(-1).astype(jnp.int32)
    x0 = _sc_gather(emb, idx)

    hf0, hr0 = _rec0(
        x0.astype(bf16).reshape(L, B, E),
        wih_0f.T.astype(bf16), wih_0r.T.astype(bf16),
        whh_0f.astype(bf16), whh_0r.astype(bf16),
        (bih_0f + bhh_0f)[None, :], (bih_0r + bhh_0r)[None, :])

    hf1, hr1 = _rec1(
        hf0, hr0,
        wih_1f[:, :H].T.astype(bf16), wih_1f[:, H:].T.astype(bf16),
        wih_1r[:, :H].T.astype(bf16), wih_1r[:, H:].T.astype(bf16),
        whh_1f.astype(bf16), whh_1r.astype(bf16),
        (bih_1f + bhh_1f)[None, :], (bih_1r + bhh_1r)[None, :])

    out, aw = _attn(hf1, hr1, wa, ba, wf, bf)
    return out, aw.reshape(B, L, 1)


# speedup vs baseline: 1.0075x; 1.0075x over previous
import functools

import jax
import jax.numpy as jnp
from jax import lax
from jax.experimental import pallas as pl
from jax.experimental.pallas import tpu as pltpu
from jax.experimental.pallas import tpu_sc as plsc

B, L, V, E, H, OUT = 64, 200, 100000, 128, 512, 2
G4 = 4 * H
TS = 10
NB = L // TS


def _sc_gather(table, idx):
    n = idx.shape[0]
    d = table.shape[1]
    info = plsc.get_sparse_core_info()
    nw = info.num_cores * info.num_subcores
    n_per_w = n // nw

    mesh = plsc.VectorSubcoreMesh(core_axis_name="c", subcore_axis_name="s")

    @functools.partial(
        pl.kernel,
        mesh=mesh,
        out_type=jax.ShapeDtypeStruct((n, d), jnp.float32),
        scratch_types=[
            pltpu.VMEM((n_per_w,), jnp.int32),
            pltpu.VMEM((n_per_w, d), jnp.float32),
            pltpu.SemaphoreType.DMA,
        ],
    )
    def gath(table_hbm, idx_hbm, out_hbm, idx_v, rows_v, sem):
        wid = lax.axis_index("s") * info.num_cores + lax.axis_index("c")
        base = wid * n_per_w
        pltpu.sync_copy(idx_hbm.at[pl.ds(base, n_per_w)], idx_v)
        pltpu.async_copy(table_hbm.at[idx_v], rows_v, sem).wait()
        pltpu.sync_copy(rows_v, out_hbm.at[pl.ds(base, n_per_w)])

    return gath(table, idx)


def _lstm_step(g_ref, wh_ref, h_s, c_s, o_ref):
    hb = h_s[...].astype(jnp.bfloat16)

    def gate(k):
        return g_ref[:, k * H:(k + 1) * H].astype(jnp.float32) + (
            lax.dot_general(hb, wh_ref[k * H:(k + 1) * H, :],
                            (((1,), (1,)), ((), ())),
                            preferred_element_type=jnp.float32))

    i = jax.nn.sigmoid(gate(0))
    f = jax.nn.sigmoid(gate(1))
    gg = jnp.tanh(gate(2))
    o = jax.nn.sigmoid(gate(3))
    c = f * c_s[...] + i * gg
    h = o * jnp.tanh(c)
    c_s[...] = c
    h_s[...] = h
    o_ref[...] = h.astype(o_ref.dtype)


def _rec0_body(xf_ref, xr_ref, wi_f_ref, wi_r_ref, whf_ref, whr_ref,
               bf_ref, br_ref, of_ref, or_ref, gf, gr, hf, cf, hr, cr):
    t = pl.program_id(0)

    @pl.when(t == 0)
    def _():
        hf[...] = jnp.zeros_like(hf)
        cf[...] = jnp.zeros_like(cf)
        hr[...] = jnp.zeros_like(hr)
        cr[...] = jnp.zeros_like(cr)

    gf[...] = (jnp.dot(xf_ref[...].reshape(TS * B, E), wi_f_ref[...],
                       preferred_element_type=jnp.float32)
               + bf_ref[...]).astype(gf.dtype)
    gr[...] = (jnp.dot(xr_ref[...].reshape(TS * B, E), wi_r_ref[...],
                       preferred_element_type=jnp.float32)
               + br_ref[...]).astype(gr.dtype)

    for j in range(TS):
        _lstm_step(gf.at[pl.ds(j * B, B)], whf_ref, hf, cf, of_ref.at[j])
        jr = TS - 1 - j
        _lstm_step(gr.at[pl.ds(jr * B, B)], whr_ref, hr, cr,
                   or_ref.at[jr])


def _rec0(x, wi_f, wi_r, whh_f, whh_r, bias_f, bias_r):
    return pl.pallas_call(
        _rec0_body,
        grid=(NB,),
        in_specs=[
            pl.BlockSpec((TS, B, E), lambda t: (t, 0, 0)),
            pl.BlockSpec((TS, B, E), lambda t: (NB - 1 - t, 0, 0)),
            pl.BlockSpec((E, G4), lambda t: (0, 0)),
            pl.BlockSpec((E, G4), lambda t: (0, 0)),
            pl.BlockSpec((G4, H), lambda t: (0, 0)),
            pl.BlockSpec((G4, H), lambda t: (0, 0)),
            pl.BlockSpec((1, G4), lambda t: (0, 0)),
            pl.BlockSpec((1, G4), lambda t: (0, 0)),
        ],
        out_specs=[
            pl.BlockSpec((TS, B, H), lambda t: (t, 0, 0)),
            pl.BlockSpec((TS, B, H), lambda t: (NB - 1 - t, 0, 0)),
        ],
        out_shape=[
            jax.ShapeDtypeStruct((L, B, H), jnp.bfloat16),
            jax.ShapeDtypeStruct((L, B, H), jnp.bfloat16),
        ],
        scratch_shapes=[
            pltpu.VMEM((TS * B, G4), jnp.bfloat16),
            pltpu.VMEM((TS * B, G4), jnp.bfloat16),
            pltpu.VMEM((B, H), jnp.float32),
            pltpu.VMEM((B, H), jnp.float32),
            pltpu.VMEM((B, H), jnp.float32),
            pltpu.VMEM((B, H), jnp.float32),
        ],
        compiler_params=pltpu.CompilerParams(
            dimension_semantics=("arbitrary",),
        ),
    )(x, x, wi_f, wi_r, whh_f, whh_r, bias_f, bias_r)


def _rec1_body(af_ref, bf2_ref, ar_ref, br2_ref,
               wia_f_ref, wib_f_ref, wia_r_ref, wib_r_ref,
               whf_ref, whr_ref, bf_ref, br_ref,
               of_ref, or_ref, gf, gr, hf, cf, hr, cr):
    t = pl.program_id(0)

    @pl.when(t == 0)
    def _():
        hf[...] = jnp.zeros_like(hf)
        cf[...] = jnp.zeros_like(cf)
        hr[...] = jnp.zeros_like(hr)
        cr[...] = jnp.zeros_like(cr)

    gf[...] = (
        jnp.dot(af_ref[...].reshape(TS * B, H), wia_f_ref[...],
                preferred_element_type=jnp.float32)
        + jnp.dot(bf2_ref[...].reshape(TS * B, H), wib_f_ref[...],
                  preferred_element_type=jnp.float32)
        + bf_ref[...]).astype(gf.dtype)
    gr[...] = (
        jnp.dot(ar_ref[...].reshape(TS * B, H), wia_r_ref[...],
                preferred_element_type=jnp.float32)
        + jnp.dot(br2_ref[...].reshape(TS * B, H), wib_r_ref[...],
                  preferred_element_type=jnp.float32)
        + br_ref[...]).astype(gr.dtype)

    for j in range(TS):
        _lstm_step(gf.at[pl.ds(j * B, B)], whf_ref, hf, cf, of_ref.at[j])
        jr = TS - 1 - j
        _lstm_step(gr.at[pl.ds(jr * B, B)], whr_ref, hr, cr,
                   or_ref.at[jr])


def _rec1(h0f, h0r, wia_f, wib_f, wia_r, wib_r, whh_f, whh_r,
          bias_f, bias_r):
    blk = pl.BlockSpec((TS, B, H), lambda t: (t, 0, 0))
    blk_rev = pl.BlockSpec((TS, B, H), lambda t: (NB - 1 - t, 0, 0))
    wspec = pl.BlockSpec((H, G4), lambda t: (0, 0))
    return pl.pallas_call(
        _rec1_body,
        grid=(NB,),
        in_specs=[
            blk, blk, blk_rev, blk_rev,
            wspec, wspec, wspec, wspec,
            pl.BlockSpec((G4, H), lambda t: (0, 0)),
            pl.BlockSpec((G4, H), lambda t: (0, 0)),
            pl.BlockSpec((1, G4), lambda t: (0, 0)),
            pl.BlockSpec((1, G4), lambda t: (0, 0)),
        ],
        out_specs=[blk, blk_rev],
        out_shape=[
            jax.ShapeDtypeStruct((L, B, H), jnp.bfloat16),
            jax.ShapeDtypeStruct((L, B, H), jnp.bfloat16),
        ],
        scratch_shapes=[
            pltpu.VMEM((TS * B, G4), jnp.bfloat16),
            pltpu.VMEM((TS * B, G4), jnp.bfloat16),
            pltpu.VMEM((B, H), jnp.float32),
            pltpu.VMEM((B, H), jnp.float32),
            pltpu.VMEM((B, H), jnp.float32),
            pltpu.VMEM((B, H), jnp.float32),
        ],
        compiler_params=pltpu.CompilerParams(
            dimension_semantics=("arbitrary",),
        ),
    )(h0f, h0r, h0f, h0r, wia_f, wib_f, wia_r, wib_r,
      whh_f, whh_r, bias_f, bias_r)


def _attn_body(xf_ref, xr_ref, wa_ref, ba_ref, wf_ref, bf_ref,
               out_ref, aw_ref):
    bb = xf_ref.shape[1]
    xf = xf_ref[...].astype(jnp.float32)
    xr = xr_ref[...].astype(jnp.float32)
    wa = wa_ref[...]
    lg = (
        jnp.dot(xf.reshape(L * bb, H), wa[:, :H].T,
                preferred_element_type=jnp.float32)
        + jnp.dot(xr.reshape(L * bb, H), wa[:, H:].T,
                  preferred_element_type=jnp.float32)
    ).reshape(L, bb) + ba_ref[0, 0]
    m = jnp.max(lg, axis=0, keepdims=True)
    e = jnp.exp(lg - m)
    w = e / jnp.sum(e, axis=0, keepdims=True)
    aw_ref[...] = w.T
    ctx_f = jnp.sum(w[:, :, None] * xf, axis=0)
    ctx_r = jnp.sum(w[:, :, None] * xr, axis=0)
    wf = wf_ref[...]
    out_ref[...] = (
        jnp.dot(ctx_f, wf[:, :H].T, preferred_element_type=jnp.float32)
        + jnp.dot(ctx_r, wf[:, H:].T, preferred_element_type=jnp.float32)
        + bf_ref[...]
    )


def _attn(h_f, h_r, wa, ba, wf, bf, bb=16):
    return pl.pallas_call(
        _attn_body,
        grid=(B // bb,),
        in_specs=[
            pl.BlockSpec((L, bb, H), lambda b: (0, b, 0)),
            pl.BlockSpec((L, bb, H), lambda b: (0, b, 0)),
            pl.BlockSpec((1, 2 * H), lambda b: (0, 0)),
            pl.BlockSpec((1, 1), lambda b: (0, 0)),
            pl.BlockSpec((OUT, 2 * H), lambda b: (0, 0)),
            pl.BlockSpec((1, OUT), lambda b: (0, 0)),
        ],
        out_specs=[
            pl.BlockSpec((bb, OUT), lambda b: (b, 0)),
            pl.BlockSpec((bb, L), lambda b: (b, 0)),
        ],
        out_shape=[
            jax.ShapeDtypeStruct((B, OUT), jnp.float32),
            jax.ShapeDtypeStruct((B, L), jnp.float32),
        ],
        compiler_params=pltpu.CompilerParams(
            dimension_semantics=("parallel",),
        ),
    )(h_f, h_r, wa, ba.reshape(1, 1), wf, bf.reshape(1, OUT))


def kernel(text, wih_0f, whh_0f, bih_0f, bhh_0f, wih_0r, whh_0r, bih_0r,
           bhh_0r, wih_1f, whh_1f, bih_1f, bhh_1f, wih_1r, whh_1r, bih_1r,
           bhh_1r, emb, wa, ba, wf, bf):
    bf16 = jnp.bfloat16
    idx = text.T.reshape(-1).astype(jnp.int32)
    x0 = _sc_gather(emb, idx)

    hf0, hr0 = _rec0(
        x0.astype(bf16).reshape(L, B, E),
        wih_0f.T.astype(bf16), wih_0r.T.astype(bf16),
        whh_0f.astype(bf16), whh_0r.astype(bf16),
        (bih_0f + bhh_0f)[None, :], (bih_0r + bhh_0r)[None, :])

    hf1, hr1 = _rec1(
        hf0, hr0,
        wih_1f[:, :H].T.astype(bf16), wih_1f[:, H:].T.astype(bf16),
        wih_1r[:, :H].T.astype(bf16), wih_1r[:, H:].T.astype(bf16),
        whh_1f.astype(bf16), whh_1r.astype(bf16),
        (bih_1f + bhh_1f)[None, :], (bih_1r + bhh_1r)[None, :])

    out, aw = _attn(hf1, hr1, wa, ba, wf, bf)
    return out, aw.reshape(B, L, 1)
